# baseline (speedup 1.0000x reference)
import functools

import jax
import jax.numpy as jnp
from jax import lax
from jax.experimental import pallas as pl
from jax.experimental.pallas import tpu as pltpu
from jax.experimental.pallas import tpu_sc as plsc

INFEATURES = 256
CAPACITY = 8192
NC, NS, L = 2, 16, 16
NW = NC * NS
R = CAPACITY // NW
NG = R // L
EPS = 1e-8


def _rsqrt(n):
    i = lax.bitcast_convert_type(n, jnp.int32)
    y = lax.bitcast_convert_type(jnp.int32(0x5F3759DF) - (i >> 1), jnp.float32)
    for _ in range(3):
        y = y * (jnp.float32(1.5) - jnp.float32(0.5) * n * y * y)
    return y


def _sc_body(x_hbm, mem_hbm, val_out, idx_out, x_v, buf, pa, pn, vb, ib):
    wid = lax.axis_index("s") * NC + lax.axis_index("c")
    _worker(wid, x_hbm, mem_hbm, val_out, idx_out, x_v, buf, pa, pn, vb, ib)


def _worker(wid, x_hbm, mem_hbm, val_out, idx_out, x_v, buf, pa, pn, vb, ib):
    base = wid * R
    pltpu.sync_copy(x_hbm, x_v)
    pltpu.sync_copy(mem_hbm.at[pl.ds(base, R)], buf)

    lane = lax.iota(jnp.int32, L)
    colbase = lane * jnp.int32(L)

    xvs = [x_v[pl.ds(L * j, L)] for j in range(INFEATURES // L)]

    def group_body(g, carry):
        bv, bi, idxv = carry
        for r16 in range(L):
            r = g * L + r16
            a0 = jnp.zeros((L,), jnp.float32)
            a1 = jnp.zeros((L,), jnp.float32)
            n0 = jnp.zeros((L,), jnp.float32)
            n1 = jnp.zeros((L,), jnp.float32)
            for j in range(INFEATURES // L):
                v = buf[r, pl.ds(L * j, L)]
                if j % 2 == 0:
                    a0 = a0 + v * xvs[j]
                    n0 = n0 + v * v
                else:
                    a1 = a1 + v * xvs[j]
                    n1 = n1 + v * v
            pa[pl.ds(r16 * L, L)] = a0 + a1
            pn[pl.ds(r16 * L, L)] = n0 + n1
        dotv = plsc.load_gather(pa, [colbase])
        nrmv = plsc.load_gather(pn, [colbase])
        for c in range(1, L):
            dotv = dotv + plsc.load_gather(pa, [colbase + c])
            nrmv = nrmv + plsc.load_gather(pn, [colbase + c])
        mn = jnp.maximum(nrmv * _rsqrt(nrmv), EPS)
        d = dotv / mn
        upd = d > bv
        bi = jnp.where(upd, idxv, bi)
        bv = jnp.where(upd, d, bv)
        return bv, bi, idxv + jnp.int32(L)

    bv0 = jnp.full((L,), -jnp.inf, jnp.float32)
    bi0 = jnp.zeros((L,), jnp.int32)
    bv, bi, _ = lax.fori_loop(0, NG, group_body, (bv0, bi0, lane))
    vb[...] = bv
    ib[...] = bi
    pltpu.sync_copy(vb, val_out.at[wid])
    pltpu.sync_copy(ib, idx_out.at[wid])


def _merge_body(x_ref, val_ref, idx_ref, out_ref):
    vals = val_ref[...]
    idxs = idx_ref[...] + lax.broadcasted_iota(jnp.int32, (NW, L), 0) * R
    m = jnp.max(vals)
    big = jnp.int32(jnp.iinfo(jnp.int32).max)
    idx = jnp.min(jnp.where(vals == m, idxs, big))
    xv = x_ref[...]
    xn = jnp.maximum(jnp.sqrt(jnp.sum(xv * xv)), jnp.float32(EPS))
    rows = lax.broadcasted_iota(jnp.int32, (64, 128), 0)
    cols = lax.broadcasted_iota(jnp.int32, (64, 128), 1)
    lin = rows * 128 + cols
    out_ref[...] = jnp.where(lin == idx, m / xn, jnp.float32(0.0))


@jax.jit
def kernel(x, memory):
    mesh = plsc.VectorSubcoreMesh(core_axis_name="c", subcore_axis_name="s")
    sc = pl.kernel(
        _sc_body,
        out_type=(
            jax.ShapeDtypeStruct((NW, L), jnp.float32),
            jax.ShapeDtypeStruct((NW, L), jnp.int32),
        ),
        mesh=mesh,
        compiler_params=pltpu.CompilerParams(needs_layout_passes=False),
        scratch_types=[
            pltpu.VMEM((INFEATURES,), jnp.float32),
            pltpu.VMEM((R, INFEATURES), jnp.float32),
            pltpu.VMEM((L * L,), jnp.float32),
            pltpu.VMEM((L * L,), jnp.float32),
            pltpu.VMEM((L,), jnp.float32),
            pltpu.VMEM((L,), jnp.int32),
        ],
    )
    cand_val, cand_idx = sc(x, memory)
    out2d = pl.pallas_call(
        _merge_body,
        out_shape=jax.ShapeDtypeStruct((64, 128), jnp.float32),
    )(x.reshape(2, 128), cand_val, cand_idx)
    return out2d.reshape(CAPACITY)
